# baseline (speedup 1.0000x reference)
import functools

import jax
import jax.numpy as jnp
from jax import lax
from jax.experimental import pallas as pl
from jax.experimental.pallas import tpu as pltpu
from jax.experimental.pallas import tpu_sc as plsc

NUM_CORES = 2
NUM_SUBCORES = 16
NW = NUM_CORES * NUM_SUBCORES


@functools.lru_cache(maxsize=None)
def _build(BATCH, HIST, V, D):
    s_per_w = BATCH // NW
    G = 16
    n_groups = s_per_w // G
    DT = D // 8
    BT = BATCH // 128
    HP = 56

    mesh = plsc.VectorSubcoreMesh(
        core_axis_name="c", subcore_axis_name="s",
        num_cores=NUM_CORES, num_subcores=NUM_SUBCORES)

    @functools.partial(
        pl.kernel,
        out_type=jax.ShapeDtypeStruct((HIST, DT, BT, 8, 128), jnp.float32),
        mesh=mesh,
        compiler_params=pltpu.CompilerParams(
            use_tc_tiling_on_sc=False, needs_layout_passes=False),
        scratch_types=[
            pltpu.VMEM((s_per_w, HP), jnp.int32),
            pltpu.VMEM((2, G, HP, D), jnp.float32),
            pltpu.VMEM((4, DT, 8, G), jnp.float32),
            pltpu.SemaphoreType.DMA,
            pltpu.SemaphoreType.DMA,
        ],
    )
    def k(idx_hbm, table_hbm, out_hbm, idx_v, rows_v, trans_v, gsem, osem):
        wid = lax.axis_index("s") * NUM_CORES + lax.axis_index("c")
        sbase = wid * s_per_w
        pltpu.sync_copy(idx_hbm.at[pl.ds(sbase, s_per_w), pl.ds(0, HP)], idx_v)

        lanes = lax.iota(jnp.int32, 16)
        dt_base = lanes // 8
        dr_vec = lanes % 8

        def gather_start(g, buf):
            for j in range(G):
                pltpu.make_async_copy(
                    table_hbm.at[idx_v.at[g * G + j]],
                    rows_v.at[buf, j], gsem).start()

        def gather_wait(g, buf):
            for j in range(G):
                pltpu.make_async_copy(
                    table_hbm.at[idx_v.at[g * G + j]],
                    rows_v.at[buf, j], gsem).wait()

        def put_desc(g, h, tbuf):
            return pltpu.make_async_copy(
                trans_v.at[tbuf],
                out_hbm.at[h, pl.ds(0, DT), wid, pl.ds(0, 8),
                           pl.ds(G * g, G)],
                osem)

        gather_start(0, 0)

        @pl.loop(0, n_groups)
        def _(g):
            buf = lax.rem(g, 2)
            gather_wait(g, buf)

            @pl.when(g + 1 < n_groups)
            def _():
                gather_start(g + 1, 1 - buf)

            @pl.loop(0, HIST)
            def _(h):
                hh = g * HIST + h
                tbuf = lax.rem(hh, 4)

                @pl.when(hh >= 4)
                def _():
                    put_desc(g, h, tbuf).wait()

                for j in range(G):
                    for kk in range(D // 16):
                        v = rows_v[buf, j, h, pl.ds(16 * kk, 16)]
                        plsc.store_scatter(
                            trans_v.at[tbuf],
                            [dt_base + 2 * kk, dr_vec,
                             jnp.full((16,), j, jnp.int32)], v)
                put_desc(g, h, tbuf).start()

        for _ in range(4):
            put_desc(0, 0, 0).wait()

    return k


def kernel(input_, table):
    batch, hist = input_.shape
    v, d = table.shape
    i32 = input_.astype(jnp.int32)
    idx = jnp.minimum(jnp.concatenate([i32, i32, i32[:, :28]], axis=1), v - 1)
    out5 = _build(batch, hist, v, d)(idx, table)
    return out5.transpose(2, 4, 0, 1, 3).reshape(batch, hist, d)


# speedup vs baseline: 1.2731x; 1.2731x over previous
import functools

import jax
import jax.numpy as jnp
from jax import lax
from jax.experimental import pallas as pl
from jax.experimental.pallas import tpu as pltpu
from jax.experimental.pallas import tpu_sc as plsc

NUM_CORES = 2
NUM_SUBCORES = 16
NW = NUM_CORES * NUM_SUBCORES


@functools.lru_cache(maxsize=None)
def _build(BATCH, HIST, V, D, K, NBUF):
    s_per_w = BATCH // NW
    n_chunks = s_per_w // K
    assert s_per_w % K == 0 and n_chunks >= NBUF
    b_per_w = s_per_w * HIST

    mesh = plsc.VectorSubcoreMesh(
        core_axis_name="c", subcore_axis_name="s",
        num_cores=NUM_CORES, num_subcores=NUM_SUBCORES)

    @functools.partial(
        pl.kernel,
        out_type=jax.ShapeDtypeStruct((BATCH, 56, 128), jnp.float32),
        mesh=mesh,
        compiler_params=pltpu.CompilerParams(use_tc_tiling_on_sc=False),
        scratch_types=[
            pltpu.VMEM((s_per_w, 56), jnp.int32),
            pltpu.VMEM((NBUF, K, 56, D), jnp.float32),
            pltpu.SemaphoreType.DMA,
            pltpu.SemaphoreType.DMA,
        ],
    )
    def k(idx_hbm, table_hbm, out_hbm, idx_v, rows_v, gsem, osem):
        wid = lax.axis_index("s") * NUM_CORES + lax.axis_index("c")
        sbase = wid * s_per_w
        pltpu.sync_copy(idx_hbm.at[pl.ds(sbase, s_per_w), pl.ds(0, 56)], idx_v)

        def gather_start(i, buf):
            for j in range(K):
                pltpu.make_async_copy(
                    table_hbm.at[idx_v.at[i * K + j]],
                    rows_v.at[buf, j], gsem).start()

        def gather_wait(i, buf):
            for j in range(K):
                pltpu.make_async_copy(
                    table_hbm.at[idx_v.at[i * K + j]],
                    rows_v.at[buf, j], gsem).wait()

        def put_desc(i, buf):
            return pltpu.make_async_copy(
                rows_v.at[buf],
                out_hbm.at[pl.ds(sbase + i * K, K), pl.ds(0, 56), pl.ds(0, D)],
                osem)

        for j in range(NBUF - 1):
            gather_start(j, j)

        @pl.loop(0, n_chunks)
        def _(i):
            buf = lax.rem(i, NBUF)
            gather_wait(i, buf)

            @pl.when(i >= 1)
            def _():
                put_desc(i - 1, lax.rem(i - 1, NBUF)).wait()

            nxt = i + NBUF - 1

            @pl.when(nxt < n_chunks)
            def _():
                gather_start(nxt, lax.rem(nxt, NBUF))

            put_desc(i, buf).start()

        put_desc(n_chunks - 1, (n_chunks - 1) % NBUF).wait()

    return k


def kernel(input_, table):
    batch, hist = input_.shape
    v, d = table.shape
    i32 = input_.astype(jnp.int32)
    idx = jnp.minimum(jnp.concatenate([i32, i32, i32[:, :28]], axis=1), v - 1)
    out = _build(batch, hist, v, d, 8, 4)(idx, table)
    return out[:, :hist, :d]
